# baseline (speedup 1.0000x reference)
import jax
import jax.numpy as jnp
from jax.experimental import pallas as pl
from jax.experimental.pallas import tpu as pltpu

_N = 4096
_GRID = 8
_BLK = _N // _GRID


def _gcn_body(x_ref, adj_ref, w1_ref, b1_ref, w2_ref, b2_ref, out_ref,
              a_cache_ref, s_ref, g_ref, acc_ref):
    t = pl.program_id(0)

    @pl.when(t == 0)
    def _init():
        s_ref[...] = jnp.dot(
            x_ref[...], w1_ref[...],
            preferred_element_type=jnp.float32).astype(jnp.bfloat16)
        g_ref[...] = jnp.zeros_like(g_ref)
        acc_ref[...] = jnp.zeros_like(acc_ref)

    @pl.when(t < _GRID)
    def _stream():
        a_cache_ref[pl.ds(t * _BLK, _BLK), :] = (
            adj_ref[...].astype(jnp.bfloat16))
        h = jnp.maximum(
            jnp.dot(a_cache_ref[pl.ds(t * _BLK, _BLK), :], s_ref[...],
                    preferred_element_type=jnp.float32)
            + b1_ref[...], 0.0)
        g_t = jnp.dot(h.astype(jnp.bfloat16), w2_ref[...],
                      preferred_element_type=jnp.float32).astype(jnp.bfloat16)
        p2 = jnp.dot(a_cache_ref[:, pl.ds(t * _BLK, _BLK)], g_t,
                     preferred_element_type=jnp.float32)
        rows = jax.lax.broadcasted_iota(jnp.int32, p2.shape, 0)
        acc_ref[...] += jnp.where(rows < t * _BLK, p2, 0.0)
        g_ref[pl.ds(t * _BLK, _BLK), :] = g_t
        acc_ref[pl.ds(t * _BLK, _BLK), :] += jnp.dot(
            a_cache_ref[pl.ds(t * _BLK, _BLK), :], g_ref[...],
            preferred_element_type=jnp.float32)

    @pl.when(t == _GRID)
    def _drain():
        o = acc_ref[...] + b2_ref[...]
        e = o - jnp.max(o, axis=1, keepdims=True)
        out_ref[...] = e - jnp.log(jnp.sum(jnp.exp(e), axis=1, keepdims=True))


def kernel(x, adj, W1, b1, W2, b2):
    n, d_in = x.shape
    d_h = W1.shape[1]
    d_out = W2.shape[1]
    b1r = b1.reshape(1, d_h)
    b2r = b2.reshape(1, d_out)

    out = pl.pallas_call(
        _gcn_body,
        grid=(_GRID + 1,),
        in_specs=[
            pl.BlockSpec((n, d_in), lambda t: (0, 0)),
            pl.BlockSpec((_BLK, n), lambda t: (jnp.minimum(t, _GRID - 1), 0)),
            pl.BlockSpec((d_in, d_h), lambda t: (0, 0)),
            pl.BlockSpec((1, d_h), lambda t: (0, 0)),
            pl.BlockSpec((d_h, d_out), lambda t: (0, 0)),
            pl.BlockSpec((1, d_out), lambda t: (0, 0)),
        ],
        out_specs=pl.BlockSpec((n, d_out), lambda t: (0, 0)),
        out_shape=jax.ShapeDtypeStruct((n, d_out), jnp.float32),
        scratch_shapes=[
            pltpu.VMEM((n, n), jnp.bfloat16),
            pltpu.VMEM((n, d_h), jnp.bfloat16),
            pltpu.VMEM((n, d_out), jnp.bfloat16),
            pltpu.VMEM((n, d_out), jnp.float32),
        ],
        compiler_params=pltpu.CompilerParams(
            vmem_limit_bytes=100 * 1024 * 1024,
        ),
    )(x, adj, W1, b1r, W2, b2r)
    return out


# speedup vs baseline: 1.0877x; 1.0877x over previous
import jax
import jax.numpy as jnp
from jax.experimental import pallas as pl
from jax.experimental.pallas import tpu as pltpu

_N = 4096
_GRID = 8
_BLK = _N // _GRID


def _gcn_body(x_ref, adj_ref, w1_ref, b1_ref, w2_ref, b2_ref, out_ref,
              a_cache_ref, s_ref, g_ref, acc_ref):
    t = pl.program_id(0)

    @pl.when(t == 0)
    def _init():
        s_ref[...] = jnp.dot(
            x_ref[...], w1_ref[...],
            preferred_element_type=jnp.float32).astype(jnp.bfloat16)
        g_ref[...] = jnp.zeros_like(g_ref)

    @pl.when(t < _GRID)
    def _stream():
        a_cache_ref[pl.ds(t * _BLK, _BLK), :] = (
            adj_ref[...].astype(jnp.bfloat16))
        h = jnp.maximum(
            jnp.dot(a_cache_ref[pl.ds(t * _BLK, _BLK), :], s_ref[...],
                    preferred_element_type=jnp.float32)
            + b1_ref[...], 0.0)
        g_t = jnp.dot(h.astype(jnp.bfloat16), w2_ref[...],
                      preferred_element_type=jnp.float32).astype(jnp.bfloat16)
        g_ref[pl.ds(t * _BLK, _BLK), :] = g_t
        acc_ref[pl.ds(t * _BLK, _BLK), :] = jnp.dot(
            a_cache_ref[pl.ds(t * _BLK, _BLK), :], g_ref[...],
            preferred_element_type=jnp.float32)

    @pl.when(t == _GRID)
    def _drain():
        for lo, mid, hi in ((0, 2048, 4096),
                            (0, 1024, 2048), (2048, 3072, 4096),
                            (0, 512, 1024), (1024, 1536, 2048),
                            (2048, 2560, 3072), (3072, 3584, 4096)):
            acc_ref[lo:mid, :] += jnp.dot(
                a_cache_ref[lo:mid, mid:hi], g_ref[mid:hi, :],
                preferred_element_type=jnp.float32)
        o = acc_ref[...] + b2_ref[...]
        e = o - jnp.max(o, axis=1, keepdims=True)
        out_ref[...] = e - jnp.log(jnp.sum(jnp.exp(e), axis=1, keepdims=True))


def kernel(x, adj, W1, b1, W2, b2):
    n, d_in = x.shape
    d_h = W1.shape[1]
    d_out = W2.shape[1]
    b1r = b1.reshape(1, d_h)
    b2r = b2.reshape(1, d_out)

    out = pl.pallas_call(
        _gcn_body,
        grid=(_GRID + 1,),
        in_specs=[
            pl.BlockSpec((n, d_in), lambda t: (0, 0)),
            pl.BlockSpec((_BLK, n), lambda t: (jnp.minimum(t, _GRID - 1), 0)),
            pl.BlockSpec((d_in, d_h), lambda t: (0, 0)),
            pl.BlockSpec((1, d_h), lambda t: (0, 0)),
            pl.BlockSpec((d_h, d_out), lambda t: (0, 0)),
            pl.BlockSpec((1, d_out), lambda t: (0, 0)),
        ],
        out_specs=pl.BlockSpec((n, d_out), lambda t: (0, 0)),
        out_shape=jax.ShapeDtypeStruct((n, d_out), jnp.float32),
        scratch_shapes=[
            pltpu.VMEM((n, n), jnp.bfloat16),
            pltpu.VMEM((n, d_h), jnp.bfloat16),
            pltpu.VMEM((n, d_out), jnp.bfloat16),
            pltpu.VMEM((n, d_out), jnp.float32),
        ],
        compiler_params=pltpu.CompilerParams(
            vmem_limit_bytes=100 * 1024 * 1024,
        ),
    )(x, adj, W1, b1r, W2, b2r)
    return out
